# Initial kernel scaffold; baseline (speedup 1.0000x reference)
import jax
import jax.numpy as jnp
from jax.experimental import pallas as pl


def kernel(miRNA, disease, mm_edge_index, dd_edge_index, md_edge_index, samples, params):
    raise NotImplementedError("write your pallas kernel here")


# speedup vs baseline: 26.6844x; 26.6844x over previous
import functools

import jax
import jax.numpy as jnp
from jax import lax
from jax.experimental import pallas as pl
from jax.experimental.pallas import tpu as pltpu
from jax.experimental.pallas import tpu_sc as plsc

_INTERPRET = False

N_M = 2048
N_D = 2048


def _elu(x):
    return jnp.where(x > 0, x, jnp.exp(jnp.minimum(x, 0.0)) - 1.0)


def _mm_body(x_ref, w_ref, b_ref, o_ref, *, act):
    acc = jnp.dot(x_ref[...], w_ref[...], preferred_element_type=jnp.float32)
    acc = acc + b_ref[...]
    if act:
        acc = _elu(acc)
    o_ref[...] = acc


def _matmul(x, w, b, act, bm=256):
    m, k = x.shape
    _, f = w.shape
    return pl.pallas_call(
        functools.partial(_mm_body, act=act),
        grid=(m // bm,),
        in_specs=[
            pl.BlockSpec((bm, k), lambda i: (i, 0)),
            pl.BlockSpec((k, f), lambda i: (0, 0)),
            pl.BlockSpec((1, f), lambda i: (0, 0)),
        ],
        out_specs=pl.BlockSpec((bm, f), lambda i: (i, 0)),
        out_shape=jax.ShapeDtypeStruct((m, f), jnp.float32),
        interpret=_INTERPRET,
    )(x, w, b.reshape(1, f))


def _proj_body(x_ref, w_ref, al_ref, ar_ref, z_ref, el_ref, er_ref):
    z = jnp.dot(x_ref[...], w_ref[...], preferred_element_type=jnp.float32)
    z_ref[...] = z
    el_ref[...] = jnp.dot(z, al_ref[...], preferred_element_type=jnp.float32)
    er_ref[...] = jnp.dot(z, ar_ref[...], preferred_element_type=jnp.float32)


def _project(x, w, al_x, ar_x, bm=256):
    m, k = x.shape
    _, f = w.shape
    return pl.pallas_call(
        _proj_body,
        grid=(m // bm,),
        in_specs=[
            pl.BlockSpec((bm, k), lambda i: (i, 0)),
            pl.BlockSpec((k, f), lambda i: (0, 0)),
            pl.BlockSpec((f, 16), lambda i: (0, 0)),
            pl.BlockSpec((f, 16), lambda i: (0, 0)),
        ],
        out_specs=[
            pl.BlockSpec((bm, f), lambda i: (i, 0)),
            pl.BlockSpec((bm, 16), lambda i: (i, 0)),
            pl.BlockSpec((bm, 16), lambda i: (i, 0)),
        ],
        out_shape=[
            jax.ShapeDtypeStruct((m, f), jnp.float32),
            jax.ShapeDtypeStruct((m, 16), jnp.float32),
            jax.ShapeDtypeStruct((m, 16), jnp.float32),
        ],
        interpret=_INTERPRET,
    )(x, w, al_x, ar_x)


def _expand_attn(a):
    h, d = a.shape
    eye = jnp.eye(h, dtype=a.dtype)
    out = (a[:, :, None] * eye[:, None, :]).reshape(h * d, h)
    return jnp.pad(out, ((0, 0), (0, 16 - h)))


def _attn_body(c_ref, z_ref, elt_ref, er_ref, b_ref, *rest, heads, dim, blend):
    if blend:
        fw_ref, res_ref, o_ref = rest
    else:
        (o_ref,) = rest
    cb = c_ref[...]
    mask = cb > 0.0
    neg = jnp.float32(-1e30)
    for h in range(heads):
        e = er_ref[:, h:h + 1] + elt_ref[h:h + 1, :]
        e = jnp.where(e >= 0, e, 0.2 * e)
        es = jnp.where(mask, e, neg)
        emax = jnp.max(es, axis=1, keepdims=True)
        emax = jnp.where(emax < -1e29, 0.0, emax)
        p = jnp.exp(es - emax) * cb
        denom = jnp.sum(p, axis=1, keepdims=True)
        o = jnp.dot(p, z_ref[:, h * dim:(h + 1) * dim],
                    preferred_element_type=jnp.float32)
        o = o / jnp.maximum(denom, 1e-9) + b_ref[:, h * dim:(h + 1) * dim]
        o = _elu(o)
        if blend:
            fw = fw_ref[...]
            o = fw * o + (1.0 - fw) * res_ref[...]
        o_ref[:, h * dim:(h + 1) * dim] = o


def _gat_attn(c, z, elt, er, b, heads, dim, fw=None, res=None, bm=256):
    n = c.shape[0]
    f = heads * dim
    blend = fw is not None
    ins = [c, z, elt, er, b.reshape(1, f)]
    in_specs = [
        pl.BlockSpec((bm, n), lambda i: (i, 0)),
        pl.BlockSpec((n, f), lambda i: (0, 0)),
        pl.BlockSpec((16, n), lambda i: (0, 0)),
        pl.BlockSpec((bm, 16), lambda i: (i, 0)),
        pl.BlockSpec((1, f), lambda i: (0, 0)),
    ]
    if blend:
        ins += [fw.reshape(1, 1), res]
        in_specs += [
            pl.BlockSpec((1, 1), lambda i: (0, 0)),
            pl.BlockSpec((bm, f), lambda i: (i, 0)),
        ]
    return pl.pallas_call(
        functools.partial(_attn_body, heads=heads, dim=dim, blend=blend),
        grid=(n // bm,),
        in_specs=in_specs,
        out_specs=pl.BlockSpec((bm, f), lambda i: (i, 0)),
        out_shape=jax.ShapeDtypeStruct((n, f), jnp.float32),
        interpret=_INTERPRET,
    )(*ins)


def _gat_layer(x, edge_c, w, al, ar, b, heads, dim, fw=None, res=None):
    al_x = _expand_attn(al)
    ar_x = _expand_attn(ar)
    z, el, er = _project(x, w, al_x, ar_x)
    elt = el.T
    bm = 128 if edge_c.shape[0] > 2048 else 256
    return _gat_attn(edge_c, z, elt, er, b, heads, dim, fw=fw, res=res, bm=bm)


def _blend_body(s_ref, a_ref, fw_ref, o_ref):
    fw = fw_ref[...]
    f = s_ref.shape[1]
    o_ref[:, :f] = fw * s_ref[...] + (1.0 - fw) * a_ref[...]
    o_ref[:, f:] = jnp.zeros_like(o_ref[:, f:])


def _blend(sim, ass, fw):
    n, f = sim.shape
    return pl.pallas_call(
        _blend_body,
        grid=(1,),
        in_specs=[
            pl.BlockSpec((n, f), lambda i: (0, 0)),
            pl.BlockSpec((n, f), lambda i: (0, 0)),
            pl.BlockSpec((1, 1), lambda i: (0, 0)),
        ],
        out_specs=pl.BlockSpec((n, 2 * f), lambda i: (0, 0)),
        out_shape=jax.ShapeDtypeStruct((n, 2 * f), jnp.float32),
        interpret=_INTERPRET,
    )(sim, ass, fw.reshape(1, 1))


def _mlp_body(g1_ref, g2_ref, w0a_ref, w0b_ref, b0_ref, w1_ref, b1_ref, o_ref):
    hh = (jnp.dot(g1_ref[...], w0a_ref[...], preferred_element_type=jnp.float32)
          + jnp.dot(g2_ref[...], w0b_ref[...], preferred_element_type=jnp.float32)
          + b0_ref[...])
    hh = _elu(hh)
    r = jnp.dot(hh, w1_ref[...], preferred_element_type=jnp.float32) + b1_ref[...]
    o_ref[...] = 1.0 / (1.0 + jnp.exp(-r))


def _mlp(g1, g2, w0, b0, w1, b1, bm=1024):
    m, f = g1.shape
    h = w0.shape[0] // 2
    w0a = jnp.pad(w0[:h], ((0, f - h), (0, 0)))
    w0b = jnp.pad(w0[h:], ((0, f - h), (0, 0)))
    return pl.pallas_call(
        _mlp_body,
        grid=(m // bm,),
        in_specs=[
            pl.BlockSpec((bm, f), lambda i: (i, 0)),
            pl.BlockSpec((bm, f), lambda i: (i, 0)),
            pl.BlockSpec((f, 64), lambda i: (0, 0)),
            pl.BlockSpec((f, 64), lambda i: (0, 0)),
            pl.BlockSpec((1, 64), lambda i: (0, 0)),
            pl.BlockSpec((64, 1), lambda i: (0, 0)),
            pl.BlockSpec((1, 1), lambda i: (0, 0)),
        ],
        out_specs=pl.BlockSpec((bm, 1), lambda i: (i, 0)),
        out_shape=jax.ShapeDtypeStruct((m, 1), jnp.float32),
        interpret=_INTERPRET,
    )(g1, g2, w0a, w0b, b0.reshape(1, 64), w1, b1.reshape(1, 1))


_USE_SC = True
_SC_MESH = dict(core_axis_name="c", subcore_axis_name="s")


def _counts_body(dst_hbm, src_hbm, zeros_hbm, out_hbm,
                 dstv, srcv, idxv, valv, nvalv, shared,
                 *, n, e, rows, tiles):
    c = lax.axis_index("c")
    s = lax.axis_index("s")
    e16 = e // 16
    sl = rows * n // 16
    wbase = s * e16
    pltpu.sync_copy(dst_hbm.at[pl.ds(wbase, e16)], dstv)
    pltpu.sync_copy(src_hbm.at[pl.ds(wbase, e16)], srcv)
    pltpu.sync_copy(zeros_hbm, shared.at[pl.ds(s * sl, sl)])
    plsc.subcore_barrier()
    for t in range(tiles):
        base = c * (n // 2) + t * rows

        @pl.loop(0, e16 // 16)
        def _(j):
            d = dstv[pl.ds(j * 16, 16)]
            sv = srcv[pl.ds(j * 16, 16)]
            m = (d >= base) & (d < base + rows)
            dc = jnp.minimum(jnp.maximum(d, base), base + rows - 1)
            off = (dc - base) * n + sv
            v = jnp.where(m, jnp.float32(1.0), jnp.float32(0.0))
            idxv[pl.ds(j * 16, 16)] = off
            valv[pl.ds(j * 16, 16)] = v
            nvalv[pl.ds(j * 16, 16)] = -v

        pltpu.sync_copy(valv, shared.at[idxv], add=True)
        plsc.subcore_barrier()
        pltpu.sync_copy(shared.at[pl.ds(s * sl, sl)],
                        out_hbm.at[pl.ds(base * n + s * sl, sl)])
        plsc.subcore_barrier()
        if t + 1 < tiles:
            pltpu.sync_copy(nvalv, shared.at[idxv], add=True)


def _build_counts_sc(edge_index, n, rows):
    e = edge_index.shape[1]
    e16 = e // 16
    tiles = (n // 2) // rows
    sl = rows * n // 16
    dst = edge_index[1]
    src = edge_index[0]
    zeros = jnp.zeros((sl,), jnp.float32)
    body = functools.partial(_counts_body, n=n, e=e, rows=rows, tiles=tiles)
    out = pl.kernel(
        body,
        out_type=jax.ShapeDtypeStruct((n * n,), jnp.float32),
        mesh=plsc.VectorSubcoreMesh(**_SC_MESH),
        scratch_types=[
            pltpu.VMEM((e16,), jnp.int32),
            pltpu.VMEM((e16,), jnp.int32),
            pltpu.VMEM((e16,), jnp.int32),
            pltpu.VMEM((e16,), jnp.float32),
            pltpu.VMEM((e16,), jnp.float32),
            pltpu.VMEM_SHARED((rows * n,), jnp.float32),
        ],
    )(dst, src, zeros)
    return out.reshape(n, n)


def _gather_body(t1_hbm, t2_hbm, s0_hbm, s1_hbm, o1_hbm, o2_hbm,
                 idxv, rowsv, sem, *, bpw):
    c = lax.axis_index("c")
    s = lax.axis_index("s")
    wid = s * 2 + c
    base = wid * bpw
    pltpu.sync_copy(s0_hbm.at[pl.ds(base, bpw)], idxv)
    pltpu.async_copy(t1_hbm.at[idxv], rowsv, sem).wait()
    pltpu.sync_copy(rowsv, o1_hbm.at[pl.ds(base, bpw)])
    pltpu.sync_copy(s1_hbm.at[pl.ds(base, bpw)], idxv)
    pltpu.async_copy(t2_hbm.at[idxv], rowsv, sem).wait()
    pltpu.sync_copy(rowsv, o2_hbm.at[pl.ds(base, bpw)])


def _gather_pairs_sc(t1, t2, s0, s1):
    b = s0.shape[0]
    d = t1.shape[1]
    bpw = b // 32
    body = functools.partial(_gather_body, bpw=bpw)
    return pl.kernel(
        body,
        out_type=[jax.ShapeDtypeStruct((b, d), jnp.float32),
                  jax.ShapeDtypeStruct((b, d), jnp.float32)],
        mesh=plsc.VectorSubcoreMesh(**_SC_MESH),
        scratch_types=[
            pltpu.VMEM((bpw,), jnp.int32),
            pltpu.VMEM((bpw, d), jnp.float32),
            pltpu.SemaphoreType.DMA,
        ],
    )(t1, t2, s0, s1)


def _build_counts(edge_index, n, rows):
    if _USE_SC:
        return _build_counts_sc(edge_index, n, rows)
    dst = edge_index[1]
    src = edge_index[0]
    return jnp.zeros((n, n), jnp.float32).at[dst, src].add(1.0)


def _gather_pairs(t1, t2, s0, s1):
    if _USE_SC:
        return _gather_pairs_sc(t1, t2, s0, s1)
    return jnp.take(t1, s0, axis=0), jnp.take(t2, s1, axis=0)


def kernel(miRNA, disease, mm_edge_index, dd_edge_index, md_edge_index,
           samples, params):
    p = params
    n_m = miRNA.shape[0]
    n_d = disease.shape[0]
    n_md = n_m + n_d

    c_mm = _build_counts(mm_edge_index, n_m, rows=512)
    c_dd = _build_counts(dd_edge_index, n_d, rows=512)
    c_md = _build_counts(md_edge_index, n_md, rows=256)

    res_mi = _matmul(miRNA, p['res_l_1_W'], p['res_l_1_b'], act=True)
    res_di = _matmul(disease, p['res_l_2_W'], p['res_l_2_b'], act=True)
    md_m = _matmul(miRNA, p['lin_m_W'], jnp.zeros((512,), jnp.float32), act=False)
    md_d = _matmul(disease, p['lin_d_W'], jnp.zeros((512,), jnp.float32), act=False)
    md = jnp.concatenate([md_m, md_d], axis=0)
    res_md = _matmul(md, p['res_l_3_W'], p['res_l_3_b'], act=True)

    h = _gat_layer(miRNA, c_mm, p['mm1_W'], p['mm1_al'], p['mm1_ar'], p['mm1_b'], 10, 128)
    h = _gat_layer(h, c_mm, p['mm2_W'], p['mm2_al'], p['mm2_ar'], p['mm2_b'], 10, 64)
    emb_mm_sim = _gat_layer(h, c_mm, p['mm3_W'], p['mm3_al'], p['mm3_ar'], p['mm3_b'],
                            1, 64, fw=p['fw_m'], res=res_mi)

    g = _gat_layer(disease, c_dd, p['dd1_W'], p['dd1_al'], p['dd1_ar'], p['dd1_b'], 10, 128)
    g = _gat_layer(g, c_dd, p['dd2_W'], p['dd2_al'], p['dd2_ar'], p['dd2_b'], 10, 64)
    emb_dd_sim = _gat_layer(g, c_dd, p['dd3_W'], p['dd3_al'], p['dd3_ar'], p['dd3_b'],
                            1, 64, fw=p['fw_d'], res=res_di)

    a = _gat_layer(md, c_md, p['md1_W'], p['md1_al'], p['md1_ar'], p['md1_b'], 10, 128)
    a = _gat_layer(a, c_md, p['md2_W'], p['md2_al'], p['md2_ar'], p['md2_b'], 10, 64)
    emb_ass = _gat_layer(a, c_md, p['md3_W'], p['md3_al'], p['md3_ar'], p['md3_b'],
                         1, 64, fw=p['fw_md'], res=res_md)

    emb_mm_ass = emb_ass[:n_m]
    emb_dd_ass = emb_ass[n_m:]

    emb_mm = _blend(emb_mm_sim, emb_mm_ass, p['fw1'])
    emb_dd = _blend(emb_dd_sim, emb_dd_ass, p['fw2'])

    s0 = jnp.asarray(samples[:, 0], jnp.int32)
    s1 = jnp.asarray(samples[:, 1], jnp.int32)
    g1, g2 = _gather_pairs(emb_mm, emb_dd, s0, s1)

    result = _mlp(g1, g2, p['mlp0_W'], p['mlp0_b'], p['mlp1_W'], p['mlp1_b'])
    return (result, emb_mm_sim, emb_mm_ass, emb_dd_sim, emb_dd_ass)


# speedup vs baseline: 33.6973x; 1.2628x over previous
import functools

import jax
import jax.numpy as jnp
from jax import lax
from jax.experimental import pallas as pl
from jax.experimental.pallas import tpu as pltpu
from jax.experimental.pallas import tpu_sc as plsc

_INTERPRET = False

N_M = 2048
N_D = 2048


def _elu(x):
    return jnp.where(x > 0, x, jnp.exp(jnp.minimum(x, 0.0)) - 1.0)


def _mm_body(x_ref, w_ref, b_ref, o_ref, *, act):
    acc = jnp.dot(x_ref[...], w_ref[...], preferred_element_type=jnp.float32)
    acc = acc + b_ref[...]
    if act:
        acc = _elu(acc)
    o_ref[...] = acc


def _matmul(x, w, b, act, bm=256):
    m, k = x.shape
    _, f = w.shape
    return pl.pallas_call(
        functools.partial(_mm_body, act=act),
        grid=(m // bm,),
        in_specs=[
            pl.BlockSpec((bm, k), lambda i: (i, 0)),
            pl.BlockSpec((k, f), lambda i: (0, 0)),
            pl.BlockSpec((1, f), lambda i: (0, 0)),
        ],
        out_specs=pl.BlockSpec((bm, f), lambda i: (i, 0)),
        out_shape=jax.ShapeDtypeStruct((m, f), jnp.float32),
        interpret=_INTERPRET,
    )(x, w, b.reshape(1, f))


def _proj_body(x_ref, w_ref, al_ref, ar_ref, z_ref, el_ref, er_ref, gl_ref, gr_ref):
    z = jnp.dot(x_ref[...], w_ref[...], preferred_element_type=jnp.float32)
    z_ref[...] = z
    el = jnp.dot(z, al_ref[...], preferred_element_type=jnp.float32)
    er = jnp.dot(z, ar_ref[...], preferred_element_type=jnp.float32)
    el_ref[...] = el
    er_ref[...] = er
    bl = jnp.broadcast_to(jnp.max(el, axis=0, keepdims=True), (8, 16))
    br = jnp.broadcast_to(jnp.max(er, axis=0, keepdims=True), (8, 16))
    i = pl.program_id(0)

    @pl.when(i == 0)
    def _():
        gl_ref[...] = bl
        gr_ref[...] = br

    @pl.when(i > 0)
    def _():
        gl_ref[...] = jnp.maximum(gl_ref[...], bl)
        gr_ref[...] = jnp.maximum(gr_ref[...], br)


def _project(x, w, al_x, ar_x, bm=256):
    m, k = x.shape
    _, f = w.shape
    return pl.pallas_call(
        _proj_body,
        grid=(m // bm,),
        in_specs=[
            pl.BlockSpec((bm, k), lambda i: (i, 0)),
            pl.BlockSpec((k, f), lambda i: (0, 0)),
            pl.BlockSpec((f, 16), lambda i: (0, 0)),
            pl.BlockSpec((f, 16), lambda i: (0, 0)),
        ],
        out_specs=[
            pl.BlockSpec((bm, f), lambda i: (i, 0)),
            pl.BlockSpec((bm, 16), lambda i: (i, 0)),
            pl.BlockSpec((bm, 16), lambda i: (i, 0)),
            pl.BlockSpec((8, 16), lambda i: (0, 0)),
            pl.BlockSpec((8, 16), lambda i: (0, 0)),
        ],
        out_shape=[
            jax.ShapeDtypeStruct((m, f), jnp.float32),
            jax.ShapeDtypeStruct((m, 16), jnp.float32),
            jax.ShapeDtypeStruct((m, 16), jnp.float32),
            jax.ShapeDtypeStruct((8, 16), jnp.float32),
            jax.ShapeDtypeStruct((8, 16), jnp.float32),
        ],
        interpret=_INTERPRET,
    )(x, w, al_x, ar_x)


def _expand_attn(a):
    h, d = a.shape
    eye = jnp.eye(h, dtype=a.dtype)
    out = (a[:, :, None] * eye[:, None, :]).reshape(h * d, h)
    return jnp.pad(out, ((0, 0), (0, 16 - h)))


def _attn_body(c_ref, z_ref, elt_ref, er_ref, gl_ref, gr_ref, b_ref, *rest,
               heads, dim, blend):
    if blend:
        fw_ref, res_ref, o_ref = rest
    else:
        (o_ref,) = rest
    cb = c_ref[...]
    g = gl_ref[0:1, :] + gr_ref[0:1, :]
    g = jnp.maximum(g, 0.2 * g)
    for h in range(heads):
        erh = er_ref[:, h:h + 1]
        gh = g[:, h:h + 1]
        col1 = erh - gh
        col2 = 0.2 * erh - gh
        row1 = elt_ref[h:h + 1, :]
        row2 = 0.2 * row1
        arg = jnp.maximum(col1 + row1, col2 + row2)
        p = jnp.exp(arg) * cb
        denom = jnp.sum(p, axis=1, keepdims=True)
        o = jnp.dot(p, z_ref[:, h * dim:(h + 1) * dim],
                    preferred_element_type=jnp.float32)
        safe = jnp.where(denom > 0.0, denom, jnp.float32(1.0))
        o = o / safe + b_ref[:, h * dim:(h + 1) * dim]
        o = _elu(o)
        if blend:
            fw = fw_ref[...]
            o = fw * o + (1.0 - fw) * res_ref[...]
        o_ref[:, h * dim:(h + 1) * dim] = o


def _gat_attn(c, z, elt, er, gl, gr, b, heads, dim, fw=None, res=None, bm=256):
    n = c.shape[0]
    f = heads * dim
    blend = fw is not None
    ins = [c, z, elt, er, gl, gr, b.reshape(1, f)]
    in_specs = [
        pl.BlockSpec((bm, n), lambda i: (i, 0)),
        pl.BlockSpec((n, f), lambda i: (0, 0)),
        pl.BlockSpec((16, n), lambda i: (0, 0)),
        pl.BlockSpec((bm, 16), lambda i: (i, 0)),
        pl.BlockSpec((8, 16), lambda i: (0, 0)),
        pl.BlockSpec((8, 16), lambda i: (0, 0)),
        pl.BlockSpec((1, f), lambda i: (0, 0)),
    ]
    if blend:
        ins += [fw.reshape(1, 1), res]
        in_specs += [
            pl.BlockSpec((1, 1), lambda i: (0, 0)),
            pl.BlockSpec((bm, f), lambda i: (i, 0)),
        ]
    return pl.pallas_call(
        functools.partial(_attn_body, heads=heads, dim=dim, blend=blend),
        grid=(n // bm,),
        in_specs=in_specs,
        out_specs=pl.BlockSpec((bm, f), lambda i: (i, 0)),
        out_shape=jax.ShapeDtypeStruct((n, f), jnp.float32),
        interpret=_INTERPRET,
    )(*ins)


def _gat_layer(x, edge_c, w, al, ar, b, heads, dim, fw=None, res=None):
    al_x = _expand_attn(al)
    ar_x = _expand_attn(ar)
    z, el, er, gl, gr = _project(x, w, al_x, ar_x)
    elt = el.T
    bm = 128 if edge_c.shape[0] > 2048 else 256
    return _gat_attn(edge_c, z, elt, er, gl, gr, b, heads, dim,
                     fw=fw, res=res, bm=bm)


def _blend_body(s_ref, a_ref, fw_ref, o_ref):
    fw = fw_ref[...]
    f = s_ref.shape[1]
    o_ref[:, :f] = fw * s_ref[...] + (1.0 - fw) * a_ref[...]
    o_ref[:, f:] = jnp.zeros_like(o_ref[:, f:])


def _blend(sim, ass, fw):
    n, f = sim.shape
    return pl.pallas_call(
        _blend_body,
        grid=(1,),
        in_specs=[
            pl.BlockSpec((n, f), lambda i: (0, 0)),
            pl.BlockSpec((n, f), lambda i: (0, 0)),
            pl.BlockSpec((1, 1), lambda i: (0, 0)),
        ],
        out_specs=pl.BlockSpec((n, 2 * f), lambda i: (0, 0)),
        out_shape=jax.ShapeDtypeStruct((n, 2 * f), jnp.float32),
        interpret=_INTERPRET,
    )(sim, ass, fw.reshape(1, 1))


def _mlp_body(g1_ref, g2_ref, w0a_ref, w0b_ref, b0_ref, w1_ref, b1_ref, o_ref):
    hh = (jnp.dot(g1_ref[...], w0a_ref[...], preferred_element_type=jnp.float32)
          + jnp.dot(g2_ref[...], w0b_ref[...], preferred_element_type=jnp.float32)
          + b0_ref[...])
    hh = _elu(hh)
    r = jnp.dot(hh, w1_ref[...], preferred_element_type=jnp.float32) + b1_ref[...]
    o_ref[...] = 1.0 / (1.0 + jnp.exp(-r))


def _mlp(g1, g2, w0, b0, w1, b1, bm=1024):
    m, f = g1.shape
    h = w0.shape[0] // 2
    w0a = jnp.pad(w0[:h], ((0, f - h), (0, 0)))
    w0b = jnp.pad(w0[h:], ((0, f - h), (0, 0)))
    return pl.pallas_call(
        _mlp_body,
        grid=(m // bm,),
        in_specs=[
            pl.BlockSpec((bm, f), lambda i: (i, 0)),
            pl.BlockSpec((bm, f), lambda i: (i, 0)),
            pl.BlockSpec((f, 64), lambda i: (0, 0)),
            pl.BlockSpec((f, 64), lambda i: (0, 0)),
            pl.BlockSpec((1, 64), lambda i: (0, 0)),
            pl.BlockSpec((64, 1), lambda i: (0, 0)),
            pl.BlockSpec((1, 1), lambda i: (0, 0)),
        ],
        out_specs=pl.BlockSpec((bm, 1), lambda i: (i, 0)),
        out_shape=jax.ShapeDtypeStruct((m, 1), jnp.float32),
        interpret=_INTERPRET,
    )(g1, g2, w0a, w0b, b0.reshape(1, 64), w1, b1.reshape(1, 1))


_USE_SC = True
_SC_MESH = dict(core_axis_name="c", subcore_axis_name="s")


def _counts_body(dst_hbm, src_hbm, zeros_hbm, out_hbm,
                 dstv, srcv, idxv, valv, nvalv, shared,
                 *, n, e, rows, tiles):
    c = lax.axis_index("c")
    s = lax.axis_index("s")
    e16 = e // 16
    sl = rows * n // 16
    wbase = s * e16
    pltpu.sync_copy(dst_hbm.at[pl.ds(wbase, e16)], dstv)
    pltpu.sync_copy(src_hbm.at[pl.ds(wbase, e16)], srcv)
    pltpu.sync_copy(zeros_hbm, shared.at[pl.ds(s * sl, sl)])
    plsc.subcore_barrier()
    for t in range(tiles):
        base = c * (n // 2) + t * rows

        @pl.loop(0, e16 // 16)
        def _(j):
            d = dstv[pl.ds(j * 16, 16)]
            sv = srcv[pl.ds(j * 16, 16)]
            m = (d >= base) & (d < base + rows)
            dc = jnp.minimum(jnp.maximum(d, base), base + rows - 1)
            off = (dc - base) * n + sv
            v = jnp.where(m, jnp.float32(1.0), jnp.float32(0.0))
            idxv[pl.ds(j * 16, 16)] = off
            valv[pl.ds(j * 16, 16)] = v
            nvalv[pl.ds(j * 16, 16)] = -v

        pltpu.sync_copy(valv, shared.at[idxv], add=True)
        plsc.subcore_barrier()
        pltpu.sync_copy(shared.at[pl.ds(s * sl, sl)],
                        out_hbm.at[pl.ds(base * n + s * sl, sl)])
        plsc.subcore_barrier()
        if t + 1 < tiles:
            pltpu.sync_copy(nvalv, shared.at[idxv], add=True)


def _build_counts_sc(edge_index, n, rows):
    e = edge_index.shape[1]
    e16 = e // 16
    tiles = (n // 2) // rows
    sl = rows * n // 16
    dst = edge_index[1]
    src = edge_index[0]
    zeros = jnp.zeros((sl,), jnp.float32)
    body = functools.partial(_counts_body, n=n, e=e, rows=rows, tiles=tiles)
    out = pl.kernel(
        body,
        out_type=jax.ShapeDtypeStruct((n * n,), jnp.float32),
        mesh=plsc.VectorSubcoreMesh(**_SC_MESH),
        scratch_types=[
            pltpu.VMEM((e16,), jnp.int32),
            pltpu.VMEM((e16,), jnp.int32),
            pltpu.VMEM((e16,), jnp.int32),
            pltpu.VMEM((e16,), jnp.float32),
            pltpu.VMEM((e16,), jnp.float32),
            pltpu.VMEM_SHARED((rows * n,), jnp.float32),
        ],
    )(dst, src, zeros)
    return out.reshape(n, n)


def _gather_body(t1_hbm, t2_hbm, s0_hbm, s1_hbm, o1_hbm, o2_hbm,
                 idxv, rowsv, sem, *, bpw):
    c = lax.axis_index("c")
    s = lax.axis_index("s")
    wid = s * 2 + c
    base = wid * bpw
    pltpu.sync_copy(s0_hbm.at[pl.ds(base, bpw)], idxv)
    pltpu.async_copy(t1_hbm.at[idxv], rowsv, sem).wait()
    pltpu.sync_copy(rowsv, o1_hbm.at[pl.ds(base, bpw)])
    pltpu.sync_copy(s1_hbm.at[pl.ds(base, bpw)], idxv)
    pltpu.async_copy(t2_hbm.at[idxv], rowsv, sem).wait()
    pltpu.sync_copy(rowsv, o2_hbm.at[pl.ds(base, bpw)])


def _gather_pairs_sc(t1, t2, s0, s1):
    b = s0.shape[0]
    d = t1.shape[1]
    bpw = b // 32
    body = functools.partial(_gather_body, bpw=bpw)
    return pl.kernel(
        body,
        out_type=[jax.ShapeDtypeStruct((b, d), jnp.float32),
                  jax.ShapeDtypeStruct((b, d), jnp.float32)],
        mesh=plsc.VectorSubcoreMesh(**_SC_MESH),
        scratch_types=[
            pltpu.VMEM((bpw,), jnp.int32),
            pltpu.VMEM((bpw, d), jnp.float32),
            pltpu.SemaphoreType.DMA,
        ],
    )(t1, t2, s0, s1)


def _build_counts(edge_index, n, rows):
    if _USE_SC:
        return _build_counts_sc(edge_index, n, rows)
    dst = edge_index[1]
    src = edge_index[0]
    return jnp.zeros((n, n), jnp.float32).at[dst, src].add(1.0)


def _gather_pairs(t1, t2, s0, s1):
    if _USE_SC:
        return _gather_pairs_sc(t1, t2, s0, s1)
    return jnp.take(t1, s0, axis=0), jnp.take(t2, s1, axis=0)


def kernel(miRNA, disease, mm_edge_index, dd_edge_index, md_edge_index,
           samples, params):
    p = params
    n_m = miRNA.shape[0]
    n_d = disease.shape[0]
    n_md = n_m + n_d

    c_mm = _build_counts(mm_edge_index, n_m, rows=512)
    c_dd = _build_counts(dd_edge_index, n_d, rows=512)
    c_md = _build_counts(md_edge_index, n_md, rows=256)

    res_mi = _matmul(miRNA, p['res_l_1_W'], p['res_l_1_b'], act=True)
    res_di = _matmul(disease, p['res_l_2_W'], p['res_l_2_b'], act=True)
    md_m = _matmul(miRNA, p['lin_m_W'], jnp.zeros((512,), jnp.float32), act=False)
    md_d = _matmul(disease, p['lin_d_W'], jnp.zeros((512,), jnp.float32), act=False)
    md = jnp.concatenate([md_m, md_d], axis=0)
    res_md = _matmul(md, p['res_l_3_W'], p['res_l_3_b'], act=True)

    h = _gat_layer(miRNA, c_mm, p['mm1_W'], p['mm1_al'], p['mm1_ar'], p['mm1_b'], 10, 128)
    h = _gat_layer(h, c_mm, p['mm2_W'], p['mm2_al'], p['mm2_ar'], p['mm2_b'], 10, 64)
    emb_mm_sim = _gat_layer(h, c_mm, p['mm3_W'], p['mm3_al'], p['mm3_ar'], p['mm3_b'],
                            1, 64, fw=p['fw_m'], res=res_mi)

    g = _gat_layer(disease, c_dd, p['dd1_W'], p['dd1_al'], p['dd1_ar'], p['dd1_b'], 10, 128)
    g = _gat_layer(g, c_dd, p['dd2_W'], p['dd2_al'], p['dd2_ar'], p['dd2_b'], 10, 64)
    emb_dd_sim = _gat_layer(g, c_dd, p['dd3_W'], p['dd3_al'], p['dd3_ar'], p['dd3_b'],
                            1, 64, fw=p['fw_d'], res=res_di)

    a = _gat_layer(md, c_md, p['md1_W'], p['md1_al'], p['md1_ar'], p['md1_b'], 10, 128)
    a = _gat_layer(a, c_md, p['md2_W'], p['md2_al'], p['md2_ar'], p['md2_b'], 10, 64)
    emb_ass = _gat_layer(a, c_md, p['md3_W'], p['md3_al'], p['md3_ar'], p['md3_b'],
                         1, 64, fw=p['fw_md'], res=res_md)

    emb_mm_ass = emb_ass[:n_m]
    emb_dd_ass = emb_ass[n_m:]

    emb_mm = _blend(emb_mm_sim, emb_mm_ass, p['fw1'])
    emb_dd = _blend(emb_dd_sim, emb_dd_ass, p['fw2'])

    s0 = jnp.asarray(samples[:, 0], jnp.int32)
    s1 = jnp.asarray(samples[:, 1], jnp.int32)
    g1, g2 = _gather_pairs(emb_mm, emb_dd, s0, s1)

    result = _mlp(g1, g2, p['mlp0_W'], p['mlp0_b'], p['mlp1_W'], p['mlp1_b'])
    return (result, emb_mm_sim, emb_mm_ass, emb_dd_sim, emb_dd_ass)
